# baseline (speedup 1.0000x reference)
import jax
import jax.numpy as jnp
from jax import lax
from jax.experimental import pallas as pl
from jax.experimental.pallas import tpu as pltpu
from jax.experimental.pallas import tpu_sc as plsc

_NC = 2
_NS = 16
_NW = _NC * _NS
_L = 16
_K = 32
_B = 16384
_BPW = _B // _NW
_NCH = 4
_CH = _BPW // _NCH
_GPC = _CH // _L


def _mf_body(x_hbm, w_hbm, h_hbm, out_hbm,
             xv, shu0, shu1, shv0, shv1, ub, vb, ov, sem0, sem1):
    sid = lax.axis_index("s")
    wid = sid * _NC + lax.axis_index("c")
    base = wid * _BPW
    pltpu.sync_copy(x_hbm.at[pl.ds(base * 2, _BPW * 2)], xv)

    shu = (shu0, shu1)
    shv = (shv0, shv1)
    sems = (sem0, sem1)
    lanes = lax.iota(jnp.int32, _L)

    def fire_chunk(c):
        su = shu[c % 2]
        sv = shv[c % 2]
        sem = sems[c % 2]

        def fire(g, carry):
            off = c * _CH * 2 + g * _L
            vec = xv[pl.ds(off, _L)]
            for j in range(_L // 2):
                s = (g * _L) // 2 + j
                pltpu.async_copy(w_hbm.at[pl.ds(vec[2 * j], 1)],
                                 su.at[sid, pl.ds(s, 1)], sem)
                pltpu.async_copy(h_hbm.at[pl.ds(vec[2 * j + 1], 1)],
                                 sv.at[sid, pl.ds(s, 1)], sem)
            return carry

        lax.fori_loop(0, 2 * _CH // _L, fire, 0)

    def drain_chunk(c):
        sem = sems[c % 2]
        pltpu.make_async_copy(w_hbm.at[pl.ds(0, _CH)], shu[c % 2].at[sid], sem).wait()
        pltpu.make_async_copy(h_hbm.at[pl.ds(0, _CH)], shv[c % 2].at[sid], sem).wait()
        pltpu.sync_copy(shu[c % 2].at[sid], ub)
        pltpu.sync_copy(shv[c % 2].at[sid], vb)

    def compute_chunk(c):
        def dot(g, carry):
            rows = g * _L + lanes
            col0 = jnp.zeros((_L,), jnp.int32)
            acc = plsc.load_gather(ub, [rows, col0]) * plsc.load_gather(vb, [rows, col0])
            for k in range(1, _K):
                colk = jnp.full((_L,), k, jnp.int32)
                acc = acc + plsc.load_gather(ub, [rows, colk]) * plsc.load_gather(vb, [rows, colk])
            ov[pl.ds(c * _CH + g * _L, _L)] = 1.0 / (1.0 + jnp.exp(-acc))
            return carry

        lax.fori_loop(0, _GPC, dot, 0)

    fire_chunk(0)
    for c in range(_NCH):
        if c + 1 < _NCH:
            fire_chunk(c + 1)
        drain_chunk(c)
        compute_chunk(c)

    pltpu.sync_copy(ov, out_hbm.at[pl.ds(base, _BPW)])


@jax.jit
def kernel(x, W, H):
    mesh = plsc.VectorSubcoreMesh(
        core_axis_name="c", subcore_axis_name="s",
        num_cores=_NC, num_subcores=_NS)
    f = pl.kernel(
        _mf_body,
        out_type=jax.ShapeDtypeStruct((_B,), jnp.float32),
        mesh=mesh,
        compiler_params=pltpu.CompilerParams(needs_layout_passes=False),
        scratch_types=[
            pltpu.VMEM((_BPW * 2,), jnp.int32),
            pltpu.VMEM_SHARED((_NS, _CH, _K), jnp.float32),
            pltpu.VMEM_SHARED((_NS, _CH, _K), jnp.float32),
            pltpu.VMEM_SHARED((_NS, _CH, _K), jnp.float32),
            pltpu.VMEM_SHARED((_NS, _CH, _K), jnp.float32),
            pltpu.VMEM((_CH, _K), jnp.float32),
            pltpu.VMEM((_CH, _K), jnp.float32),
            pltpu.VMEM((_BPW,), jnp.float32),
            pltpu.SemaphoreType.DMA,
            pltpu.SemaphoreType.DMA,
        ],
    )
    return f(x.reshape(-1), W, H)


# speedup vs baseline: 1.0923x; 1.0923x over previous
import jax
import jax.numpy as jnp
from jax import lax
from jax.experimental import pallas as pl
from jax.experimental.pallas import tpu as pltpu
from jax.experimental.pallas import tpu_sc as plsc

_NC = 2
_NS = 16
_NW = _NC * _NS
_L = 16
_K = 32
_B = 16384
_BPW = _B // _NW
_NCH = 4
_CH = _BPW // _NCH
_GPC = _CH // _L


def _mf_body(x_hbm, w_hbm, h_hbm, out_hbm,
             xv, uv0, uv1, vv0, vv1, ov, *sems):
    wid = lax.axis_index("s") * _NC + lax.axis_index("c")
    base = wid * _BPW
    pltpu.sync_copy(x_hbm.at[pl.ds(base * 2, _BPW * 2)], xv)

    ubufs = (uv0, uv1)
    vbufs = (vv0, vv1)
    lanes = lax.iota(jnp.int32, _L)

    def fire_chunk(c):
        ub = ubufs[c % 2]
        vb = vbufs[c % 2]
        sgrp = sems[(c % 2) * 8:(c % 2) * 8 + 8]

        def fire(g, carry):
            off = c * _CH * 2 + g * _L
            vec = xv[pl.ds(off, _L)]
            for j in range(_L // 2):
                s = (g * _L) // 2 + j
                pltpu.async_copy(w_hbm.at[pl.ds(vec[2 * j], 1)],
                                 ub.at[pl.ds(s, 1)], sgrp[j])
                pltpu.async_copy(h_hbm.at[pl.ds(vec[2 * j + 1], 1)],
                                 vb.at[pl.ds(s, 1)], sgrp[j])
            return carry

        lax.fori_loop(0, 2 * _CH // _L, fire, 0)

    def drain_chunk(c):
        sgrp = sems[(c % 2) * 8:(c % 2) * 8 + 8]
        for j in range(8):
            pltpu.make_async_copy(w_hbm.at[pl.ds(0, 32)],
                                  ubufs[c % 2].at[pl.ds(0, 32)], sgrp[j]).wait()

    def compute_chunk(c):
        ub = ubufs[c % 2]
        vb = vbufs[c % 2]

        def dot(g, carry):
            rows = g * _L + lanes
            col0 = jnp.zeros((_L,), jnp.int32)
            acc = plsc.load_gather(ub, [rows, col0]) * plsc.load_gather(vb, [rows, col0])
            for k in range(1, _K):
                colk = jnp.full((_L,), k, jnp.int32)
                acc = acc + plsc.load_gather(ub, [rows, colk]) * plsc.load_gather(vb, [rows, colk])
            ov[pl.ds(c * _CH + g * _L, _L)] = 1.0 / (1.0 + jnp.exp(-acc))
            return carry

        lax.fori_loop(0, _GPC, dot, 0)

    fire_chunk(0)
    for c in range(_NCH):
        if c + 1 < _NCH:
            fire_chunk(c + 1)
        drain_chunk(c)
        compute_chunk(c)

    pltpu.sync_copy(ov, out_hbm.at[pl.ds(base, _BPW)])


@jax.jit
def kernel(x, W, H):
    mesh = plsc.VectorSubcoreMesh(
        core_axis_name="c", subcore_axis_name="s",
        num_cores=_NC, num_subcores=_NS)
    f = pl.kernel(
        _mf_body,
        out_type=jax.ShapeDtypeStruct((_B,), jnp.float32),
        mesh=mesh,
        compiler_params=pltpu.CompilerParams(needs_layout_passes=False),
        scratch_types=[
            pltpu.VMEM((_BPW * 2,), jnp.int32),
            pltpu.VMEM((_CH, _K), jnp.float32),
            pltpu.VMEM((_CH, _K), jnp.float32),
            pltpu.VMEM((_CH, _K), jnp.float32),
            pltpu.VMEM((_CH, _K), jnp.float32),
            pltpu.VMEM((_BPW,), jnp.float32),
        ] + [pltpu.SemaphoreType.DMA] * 16,
    )
    return f(x.reshape(-1), W, H)
